# Initial kernel scaffold; baseline (speedup 1.0000x reference)
import jax
import jax.numpy as jnp
from jax.experimental import pallas as pl


def kernel(x, edge_index, edge_weight, W0, W1):
    raise NotImplementedError("write your pallas kernel here")


# speedup vs baseline: 4.8744x; 4.8744x over previous
import functools

import jax
import jax.numpy as jnp
from jax import lax
from jax.experimental import pallas as pl
from jax.experimental.pallas import tpu as pltpu
from jax.experimental.pallas import tpu_sc as plsc

NC = 2
NS = 16
NW = NC * NS
LANES = 16
EBLK = 128
KCH = 8

_GDN = lax.GatherDimensionNumbers(
    offset_dims=(), collapsed_slice_dims=(0,), start_index_map=(0,))


def _vgather(x, idx):
    return lax.gather(x, idx[:, None], _GDN, (1,),
                      mode=lax.GatherScatterMode.PROMISE_IN_BOUNDS)


def _spmm_sc(n, e, d):
    rows = e // EBLK
    c8 = rows // KCH
    rem = rows - c8 * KCH
    base_ch = c8 // NW
    extra = c8 % NW
    nz_a = (n // NS) & ~7
    nz_last = n - (NS - 1) * nz_a
    cd = d // LANES
    mesh = plsc.VectorSubcoreMesh(core_axis_name="c", subcore_axis_name="s")

    @functools.partial(
        pl.kernel,
        out_type=jax.ShapeDtypeStruct((NC, n, d), jnp.float32),
        mesh=mesh,
        scratch_types=[
            pltpu.VMEM((KCH, EBLK), jnp.int32),
            pltpu.VMEM((KCH, EBLK), jnp.int32),
            pltpu.VMEM((KCH, EBLK), jnp.float32),
            pltpu.VMEM((KCH * EBLK, d), jnp.float32),
            pltpu.VMEM_SHARED((n, d), jnp.float32),
            pltpu.SemaphoreType.DMA,
        ],
        compiler_params=pltpu.CompilerParams(use_tc_tiling_on_sc=False),
    )
    def spmm(src_hbm, dst_hbm, w_hbm, tbl_hbm, out_hbm,
             src_v, dst_v, w_v, rows_v, accum, sem):
        cid = lax.axis_index("c")
        sid = lax.axis_index("s")
        wid = sid * NC + cid

        zero16 = jnp.zeros((LANES,), jnp.float32)

        def zrow(r, _):
            for k in range(cd):
                rows_v[r, pl.ds(k * LANES, LANES)] = zero16
            return 0

        lax.fori_loop(0, max(nz_a, nz_last), zrow, 0)

        @pl.when(sid < NS - 1)
        def _():
            pltpu.sync_copy(rows_v.at[pl.ds(0, nz_a)],
                            accum.at[pl.ds(sid * nz_a, nz_a)])

        @pl.when(sid == NS - 1)
        def _():
            pltpu.sync_copy(rows_v.at[pl.ds(0, nz_last)],
                            accum.at[pl.ds((NS - 1) * nz_a, nz_last)])

        plsc.subcore_barrier()

        def do_chunk(row0, k):
            pltpu.sync_copy(src_hbm.at[pl.ds(row0, k)], src_v.at[pl.ds(0, k)])
            pltpu.sync_copy(dst_hbm.at[pl.ds(row0, k)], dst_v.at[pl.ds(0, k)])
            pltpu.sync_copy(w_hbm.at[pl.ds(row0, k)], w_v.at[pl.ds(0, k)])
            descs = [
                pltpu.async_copy(tbl_hbm.at[src_v.at[j]],
                                 rows_v.at[pl.ds(j * EBLK, EBLK)], sem)
                for j in range(k)
            ]
            for dsc in descs:
                dsc.wait()

            for j in range(k):
                def scale_group(g, _, j=j):
                    wv = w_v[j, pl.ds(g * LANES, LANES)]
                    for r in range(LANES):
                        wb = _vgather(wv, jnp.full((LANES,), r, jnp.int32))
                        row = j * EBLK + g * LANES + r
                        for c in range(cd):
                            sl = pl.ds(c * LANES, LANES)
                            rows_v[row, sl] = rows_v[row, sl] * wb
                    return 0

                lax.fori_loop(0, EBLK // LANES, scale_group, 0)
            for j in range(k):
                pltpu.sync_copy(rows_v.at[pl.ds(j * EBLK, EBLK)],
                                accum.at[dst_v.at[j]], add=True)

        def chunk_body(ci, _):
            do_chunk((ci * NW + wid) * KCH, KCH)
            return 0

        lax.fori_loop(0, base_ch, chunk_body, 0)
        if extra:
            @pl.when(wid < extra)
            def _():
                do_chunk((base_ch * NW + wid) * KCH, KCH)
        if rem:
            @pl.when(wid == extra)
            def _():
                do_chunk(c8 * KCH, rem)

        plsc.subcore_barrier()

        @pl.when(sid < NS - 1)
        def _():
            pltpu.sync_copy(accum.at[pl.ds(sid * nz_a, nz_a)],
                            out_hbm.at[cid, pl.ds(sid * nz_a, nz_a)])

        @pl.when(sid == NS - 1)
        def _():
            pltpu.sync_copy(accum.at[pl.ds((NS - 1) * nz_a, nz_last)],
                            out_hbm.at[cid, pl.ds((NS - 1) * nz_a, nz_last)])

    return spmm


def _mm_first(x, w0):
    n, dx = x.shape
    h = w0.shape[1]
    blk = 2000

    def body(x_ref, w_ref, o_ref):
        o_ref[...] = jnp.dot(x_ref[...], w_ref[...],
                             preferred_element_type=jnp.float32)

    return pl.pallas_call(
        body,
        grid=(n // blk,),
        in_specs=[
            pl.BlockSpec((blk, dx), lambda i: (i, 0)),
            pl.BlockSpec((dx, h), lambda i: (0, 0)),
        ],
        out_specs=pl.BlockSpec((blk, h), lambda i: (i, 0)),
        out_shape=jax.ShapeDtypeStruct((n, h), jnp.float32),
    )(x, w0)


def _relu_mm(p, w1):
    _, n, d = p.shape
    h = w1.shape[1]
    blk = 2000

    def body(p_ref, w_ref, o_ref):
        hb = jnp.maximum(p_ref[0] + p_ref[1], 0.0)
        o_ref[...] = jnp.dot(hb, w_ref[...],
                             preferred_element_type=jnp.float32)

    return pl.pallas_call(
        body,
        grid=(n // blk,),
        in_specs=[
            pl.BlockSpec((2, blk, d), lambda i: (0, i, 0)),
            pl.BlockSpec((d, h), lambda i: (0, 0)),
        ],
        out_specs=pl.BlockSpec((blk, h), lambda i: (i, 0)),
        out_shape=jax.ShapeDtypeStruct((n, h), jnp.float32),
    )(p, w1)


def _decode(q):
    _, n, d = q.shape
    blk = 200

    def body(qi_ref, qf_ref, o_ref):
        zi = qi_ref[0] + qi_ref[1]
        zf = qf_ref[0] + qf_ref[1]
        o_ref[...] = lax.dot_general(zi, zf, (((1,), (1,)), ((), ())),
                                     preferred_element_type=jnp.float32)

    return pl.pallas_call(
        body,
        grid=(n // blk,),
        in_specs=[
            pl.BlockSpec((2, blk, d), lambda i: (0, i, 0)),
            pl.BlockSpec((2, n, d), lambda i: (0, 0, 0)),
        ],
        out_specs=pl.BlockSpec((blk, n), lambda i: (i, 0)),
        out_shape=jax.ShapeDtypeStruct((n, n), jnp.float32),
    )(q, q)


def kernel(x, edge_index, edge_weight, W0, W1):
    n = x.shape[0]
    e = edge_weight.shape[0]
    src, dst, w = edge_index[0], edge_index[1], edge_weight
    pad = (-e) % (KCH * EBLK)
    if pad:
        src = jnp.concatenate([src, jnp.zeros((pad,), src.dtype)])
        dst = jnp.concatenate([dst, jnp.zeros((pad,), dst.dtype)])
        w = jnp.concatenate([w, jnp.zeros((pad,), w.dtype)])
        e += pad
    src2 = src.reshape(e // EBLK, EBLK)
    dst2 = dst.reshape(e // EBLK, EBLK)
    w2 = w.reshape(e // EBLK, EBLK)

    xw = _mm_first(x, W0)
    p = _spmm_sc(n, e, W0.shape[1])(src2, dst2, w2, xw)
    hw = _relu_mm(p, W1)
    q = _spmm_sc(n, e, W1.shape[1])(src2, dst2, w2, hw)
    out = _decode(q)
    return out.reshape(-1)
